# baseline (speedup 1.0000x reference)
import functools

import numpy as np
import jax
import jax.numpy as jnp
from jax import lax
from jax.experimental import pallas as pl
from jax.experimental.pallas import tpu as pltpu
from jax.experimental.pallas import tpu_sc as plsc

_L = 16
_T = 2 ** 14
_MASK = _T - 1
_P1L = np.int32(2654435761 % _T)
_P2L = np.int32(805459861 % _T)

_NC, _NS, _LANES = 2, 16, 16
_NW = _NC * _NS

_BATCH, _NPTS = 4096, 128
_P = _BATCH * _NPTS
_PT = _P // _NW
_BP = 2048
_NB = _PT // _BP
_VPB = _BP // _LANES


def _level_consts():
    n_min, n_max = 16.0, 512.0
    growth = float(np.exp((np.log(n_max) - np.log(n_min)) / (_L - 1)))
    levels = []
    for i in range(_L):
        n = float(np.floor(n_min * growth ** i))
        c64 = np.float64(2.0) / np.float64(n)
        is_pow2 = (n == 2.0 ** round(np.log2(n)))
        m, e = np.frexp(c64)
        c_h = np.ldexp(np.floor(np.ldexp(m, 14)), int(e) - 14)
        levels.append(dict(
            half_n=np.float32(n / 2.0),
            recip=np.float32(1.0 / c64),
            c_f=np.float32(c64),
            c_h=np.float32(c_h),
            c_l=np.float32(c64 - c_h),
            is_pow2=is_pow2,
        ))
    return levels


_LEVELS = _level_consts()

_F1 = np.float32(1.0)
_F0 = np.float32(0.0)
_I1 = np.int32(1)


def _index_and_weight(x1, lc):
    if lc["is_pow2"]:
        s = x1 * lc["half_n"]
        idx = s.astype(jnp.int32)
        w = s - idx.astype(jnp.float32)
        return idx, w
    s = x1 * lc["recip"]
    n0 = s.astype(jnp.int32)
    f0 = n0.astype(jnp.float32)
    r0 = (x1 - f0 * lc["c_h"]) - f0 * lc["c_l"]
    su = jnp.where(r0 >= lc["c_f"], _F1, _F0)
    sd = jnp.where(r0 < _F0, _F1, _F0)
    fi = (f0 + su) - sd
    idx = fi.astype(jnp.int32)
    rw = (r0 - su * lc["c_f"]) + sd * lc["c_f"]
    w = rw * lc["recip"]
    return idx, w


_SIGN = np.int32(-2147483648)
_EXPM = np.int32(0xFF)
_MANM = np.int32(0x7FFFFF)
_I0 = np.int32(0)


def _f64_words(z):
    b = plsc.bitcast(z, jnp.int32)
    exp = (b >> np.int32(23)) & _EXPM
    man = b & _MANM
    sign = b & _SIGN
    nz = exp != _I0
    hi = sign | jnp.where(nz, ((exp + np.int32(896)) << np.int32(20))
                          | (man >> np.int32(3)), _I0)
    lo = jnp.where(nz, man << np.int32(29), _I0)
    return lo, hi


def _sc_body(x_hbm, tab_hbm, out_hbm, xv0, xv1, xv2, out_v, tab_v):
    wid = (lax.axis_index("s") * _NC + lax.axis_index("c")).astype(jnp.int32)
    base = wid * np.int32(_PT)
    iota32 = jnp.arange(_LANES, dtype=jnp.int32) * np.int32(2 * _L)

    def block_body(b, carry):
        pt0 = base + b * np.int32(_BP)
        pltpu.sync_copy(x_hbm.at[pl.ds(pt0, _BP)], xv0)
        pltpu.sync_copy(x_hbm.at[pl.ds(pt0 + np.int32(_P), _BP)], xv1)
        pltpu.sync_copy(x_hbm.at[pl.ds(pt0 + np.int32(2 * _P), _BP)], xv2)
        for l in range(_L):
            lc = _LEVELS[l]
            pltpu.sync_copy(tab_hbm.at[pl.ds(np.int32(l * _T), _T)], tab_v)

            def vec_body(i, carry, lc=lc, l=l):
                off = i * np.int32(_LANES)
                xx = xv0[pl.ds(off, _LANES)]
                xy = xv1[pl.ds(off, _LANES)]
                xz = xv2[pl.ds(off, _LANES)]
                ix, wx = _index_and_weight(xx + _F1, lc)
                iy, wy = _index_and_weight(xy + _F1, lc)
                iz, wz = _index_and_weight(xz + _F1, lc)
                hy = iy * _P1L
                hz = iz * _P2L
                xa = ix
                xb = ix + _I1
                ya = hy & _MASK
                yb = (hy + _P1L) & _MASK
                za = hz & _MASK
                zb = (hz + _P2L) & _MASK
                p00 = ya ^ za
                p01 = ya ^ zb
                p10 = yb ^ za
                p11 = yb ^ zb
                v0 = plsc.load_gather(tab_v, [xa ^ p00])
                v1 = plsc.load_gather(tab_v, [xa ^ p01])
                v2 = plsc.load_gather(tab_v, [xa ^ p10])
                v3 = plsc.load_gather(tab_v, [xa ^ p11])
                v4 = plsc.load_gather(tab_v, [xb ^ p00])
                v5 = plsc.load_gather(tab_v, [xb ^ p01])
                v6 = plsc.load_gather(tab_v, [xb ^ p10])
                v7 = plsc.load_gather(tab_v, [xb ^ p11])
                x0 = v0 + wx * (v4 - v0)
                x1 = v1 + wx * (v5 - v1)
                x2 = v2 + wx * (v6 - v2)
                x3 = v3 + wx * (v7 - v3)
                y0 = x0 + wy * (x2 - x0)
                y1 = x1 + wy * (x3 - x1)
                z = y0 + wz * (y1 - y0)
                lo, hi = _f64_words(z)
                fidx = iota32 + (i * np.int32(2 * _LANES * _L)
                                 + np.int32(2 * l))
                plsc.store_scatter(out_v, [fidx], lo)
                plsc.store_scatter(out_v, [fidx + _I1], hi)
                return carry

            lax.fori_loop(jnp.int32(0), jnp.int32(_VPB), vec_body,
                          jnp.int32(0))
        pltpu.sync_copy(out_v, out_hbm.at[pl.ds(pt0 * np.int32(2 * _L),
                                                _BP * 2 * _L)])
        return carry

    lax.fori_loop(jnp.int32(0), jnp.int32(_NB), block_body, jnp.int32(0))


@functools.cache
def _hash_encode_sc():
    return pl.kernel(
        _sc_body,
        out_type=jax.ShapeDtypeStruct((_P * _L * 2,), jnp.int32),
        mesh=plsc.VectorSubcoreMesh(core_axis_name="c", subcore_axis_name="s",
                                    num_cores=_NC, num_subcores=_NS),
        scratch_types=[
            pltpu.VMEM((_BP,), jnp.float32),
            pltpu.VMEM((_BP,), jnp.float32),
            pltpu.VMEM((_BP,), jnp.float32),
            pltpu.VMEM((_BP * _L * 2,), jnp.int32),
            pltpu.VMEM((_T,), jnp.float32),
        ],
        compiler_params=pltpu.CompilerParams(needs_layout_passes=False),
    )


def kernel(x, tables):
    xt = x.reshape(_P, 3).T.reshape(3 * _P)
    tab = tables.reshape(_L * _T)
    out = _hash_encode_sc()(xt, tab)
    pairs = out.reshape(_BATCH, _NPTS, _L, 2)
    return lax.bitcast_convert_type(pairs, jnp.float64)


# speedup vs baseline: 4.0748x; 4.0748x over previous
import functools

import numpy as np
import jax
import jax.numpy as jnp
from jax import lax
from jax.experimental import pallas as pl
from jax.experimental.pallas import tpu as pltpu
from jax.experimental.pallas import tpu_sc as plsc

_L = 16
_T = 2 ** 14
_MASK = _T - 1
_P1L = np.int32(2654435761 % _T)
_P2L = np.int32(805459861 % _T)

_NC, _NS, _LANES = 2, 16, 16
_NW = _NC * _NS

_BATCH, _NPTS = 4096, 128
_P = _BATCH * _NPTS
_PT = _P // _NW
_BP = 4096
_NB = _PT // _BP
_VPB = _BP // _LANES


def _level_consts():
    n_min, n_max = 16.0, 512.0
    growth = float(np.exp((np.log(n_max) - np.log(n_min)) / (_L - 1)))
    levels = []
    for i in range(_L):
        n = float(np.floor(n_min * growth ** i))
        c64 = np.float64(2.0) / np.float64(n)
        is_pow2 = (n == 2.0 ** round(np.log2(n)))
        m, e = np.frexp(c64)
        c_h = np.ldexp(np.floor(np.ldexp(m, 14)), int(e) - 14)
        levels.append(dict(
            half_n=np.float32(n / 2.0),
            recip=np.float32(1.0 / c64),
            c_f=np.float32(c64),
            c_h=np.float32(c_h),
            c_l=np.float32(c64 - c_h),
            is_pow2=is_pow2,
        ))
    return levels


_LEVELS = _level_consts()

_F1 = np.float32(1.0)
_F0 = np.float32(0.0)
_I1 = np.int32(1)


def _index_and_weight(x1, lc):
    if lc["is_pow2"]:
        s = x1 * lc["half_n"]
        idx = s.astype(jnp.int32)
        w = s - idx.astype(jnp.float32)
        return idx, w
    s = x1 * lc["recip"]
    n0 = s.astype(jnp.int32)
    f0 = n0.astype(jnp.float32)
    r0 = (x1 - f0 * lc["c_h"]) - f0 * lc["c_l"]
    su = jnp.where(r0 >= lc["c_f"], _F1, _F0)
    sd = jnp.where(r0 < _F0, _F1, _F0)
    fi = (f0 + su) - sd
    idx = fi.astype(jnp.int32)
    rw = (r0 - su * lc["c_f"]) + sd * lc["c_f"]
    w = rw * lc["recip"]
    return idx, w


def _sc_body(x_hbm, tab_hbm, out_hbm, xv0, xv1, xv2, out_v, tab_a, tab_b,
             sem_a, sem_b):
    wid = (lax.axis_index("s") * _NC + lax.axis_index("c")).astype(jnp.int32)
    base = wid * np.int32(_PT)
    iota16 = jnp.arange(_LANES, dtype=jnp.int32) * np.int32(_L)
    tabs = (tab_a, tab_b)
    sems = (sem_a, sem_b)

    def tab_copy(l):
        return pltpu.make_async_copy(
            tab_hbm.at[pl.ds(np.int32(l * _T), _T)], tabs[l % 2],
            sems[l % 2])

    tab_copy(0).start()

    def block_body(b, carry):
        pt0 = base + b * np.int32(_BP)
        pltpu.sync_copy(x_hbm.at[pl.ds(pt0, _BP)], xv0)
        pltpu.sync_copy(x_hbm.at[pl.ds(pt0 + np.int32(_P), _BP)], xv1)
        pltpu.sync_copy(x_hbm.at[pl.ds(pt0 + np.int32(2 * _P), _BP)], xv2)
        for l in range(_L):
            lc = _LEVELS[l]
            tab_copy(l).wait()
            tab_copy((l + 1) % _L).start()
            tab_v = tabs[l % 2]

            def vec_body(i, carry, lc=lc, l=l, tab_v=tab_v):
                off = i * np.int32(_LANES)
                xx = xv0[pl.ds(off, _LANES)]
                xy = xv1[pl.ds(off, _LANES)]
                xz = xv2[pl.ds(off, _LANES)]
                ix, wx = _index_and_weight(xx + _F1, lc)
                iy, wy = _index_and_weight(xy + _F1, lc)
                iz, wz = _index_and_weight(xz + _F1, lc)
                hy = iy * _P1L
                hz = iz * _P2L
                xa = ix
                xb = ix + _I1
                ya = hy & _MASK
                yb = (hy + _P1L) & _MASK
                za = hz & _MASK
                zb = (hz + _P2L) & _MASK
                p00 = ya ^ za
                p01 = ya ^ zb
                p10 = yb ^ za
                p11 = yb ^ zb
                v0 = plsc.load_gather(tab_v, [xa ^ p00])
                v1 = plsc.load_gather(tab_v, [xa ^ p01])
                v2 = plsc.load_gather(tab_v, [xa ^ p10])
                v3 = plsc.load_gather(tab_v, [xa ^ p11])
                v4 = plsc.load_gather(tab_v, [xb ^ p00])
                v5 = plsc.load_gather(tab_v, [xb ^ p01])
                v6 = plsc.load_gather(tab_v, [xb ^ p10])
                v7 = plsc.load_gather(tab_v, [xb ^ p11])
                x0 = v0 + wx * (v4 - v0)
                x1 = v1 + wx * (v5 - v1)
                x2 = v2 + wx * (v6 - v2)
                x3 = v3 + wx * (v7 - v3)
                y0 = x0 + wy * (x2 - x0)
                y1 = x1 + wy * (x3 - x1)
                z = y0 + wz * (y1 - y0)
                fidx = iota16 + (i * np.int32(_LANES * _L) + np.int32(l))
                plsc.store_scatter(out_v, [fidx], z)
                return carry

            lax.fori_loop(jnp.int32(0), jnp.int32(_VPB), vec_body,
                          jnp.int32(0))
        pltpu.sync_copy(out_v, out_hbm.at[pl.ds(pt0 * np.int32(_L),
                                                _BP * _L)])
        return carry

    lax.fori_loop(jnp.int32(0), jnp.int32(_NB), block_body, jnp.int32(0))
    tab_copy(0).wait()


@functools.cache
def _hash_encode_sc():
    return pl.kernel(
        _sc_body,
        out_type=jax.ShapeDtypeStruct((_P * _L,), jnp.float32),
        mesh=plsc.VectorSubcoreMesh(core_axis_name="c", subcore_axis_name="s",
                                    num_cores=_NC, num_subcores=_NS),
        scratch_types=[
            pltpu.VMEM((_BP,), jnp.float32),
            pltpu.VMEM((_BP,), jnp.float32),
            pltpu.VMEM((_BP,), jnp.float32),
            pltpu.VMEM((_BP * _L,), jnp.float32),
            pltpu.VMEM((_T,), jnp.float32),
            pltpu.VMEM((_T,), jnp.float32),
            pltpu.SemaphoreType.DMA,
            pltpu.SemaphoreType.DMA,
        ],
        compiler_params=pltpu.CompilerParams(needs_layout_passes=False),
    )


def _f32_to_f64_bits(v):
    b = lax.bitcast_convert_type(v, jnp.uint32)
    exp = (b >> np.uint32(23)) & np.uint32(0xFF)
    man = b & np.uint32(0x7FFFFF)
    sign = b & np.uint32(0x80000000)
    nz = exp != np.uint32(0)
    hi = sign | jnp.where(nz, ((exp + np.uint32(896)) << np.uint32(20))
                          | (man >> np.uint32(3)), np.uint32(0))
    lo = jnp.where(nz, man << np.uint32(29), np.uint32(0))
    pairs = jnp.stack([lo, hi], axis=-1)
    return lax.bitcast_convert_type(pairs, jnp.float64)


def kernel(x, tables):
    xt = x.reshape(_P, 3).T.reshape(3 * _P)
    tab = tables.reshape(_L * _T)
    out = _hash_encode_sc()(xt, tab)
    return _f32_to_f64_bits(out.reshape(_BATCH, _NPTS, _L))
